# Initial kernel scaffold; baseline (speedup 1.0000x reference)
import jax
import jax.numpy as jnp
from jax.experimental import pallas as pl


def kernel(X, table):
    raise NotImplementedError("write your pallas kernel here")


# speedup vs baseline: 3.9340x; 3.9340x over previous
import functools

import jax
import jax.numpy as jnp
from jax import lax
from jax.experimental import pallas as pl
from jax.experimental.pallas import tpu as pltpu
from jax.experimental.pallas import tpu_sc as plsc

VOCAB = 100000
EMBED = 128
MAX_LEN = 512
BATCH = 1024
SEQ = 200
SCALE = float(EMBED) ** 0.5

NC, NS = 2, 16
NW = NC * NS
TOKENS = BATCH * SEQ
TOK_PER_W = TOKENS // NW
SEQ_PER_W = TOK_PER_W // SEQ
CHUNK = 40
NCHUNK = SEQ // CHUNK
LANES = 16


def _pe_slice():
    x = jnp.arange(MAX_LEN, dtype=jnp.float32).reshape((-1, 1)) / jnp.power(
        10000.0, jnp.arange(0, EMBED, 2, dtype=jnp.float32) / EMBED)
    pe = jnp.zeros((MAX_LEN, EMBED), dtype=jnp.float32)
    pe = pe.at[:, 0::2].set(jnp.sin(x))
    pe = pe.at[:, 1::2].set(jnp.sin(x))
    return pe[:SEQ]


def _emb_body(x_hbm, table_hbm, pe_hbm, out_hbm, pe_v, idx_v, rows_v, sem):
    wid = lax.axis_index("s") * NC + lax.axis_index("c")
    base = wid * TOK_PER_W
    pltpu.sync_copy(pe_hbm, pe_v)

    def seq_body(i, carry):
        tok = base + i * SEQ
        pltpu.sync_copy(x_hbm.at[pl.ds(tok, SEQ)], idx_v)
        copies = []
        for c in range(NCHUNK):
            copies.append(pltpu.async_copy(
                table_hbm.at[idx_v.at[pl.ds(c * CHUNK, CHUNK)]],
                rows_v.at[pl.ds(c * CHUNK, CHUNK)], sem))
        for cp in copies:
            cp.wait()

        def tok_body(t, tc):
            for d in range(EMBED // LANES):
                s = pl.ds(d * LANES, LANES)
                rows_v[t, s] = rows_v[t, s] * SCALE + pe_v[t, s]
            return tc
        lax.fori_loop(0, SEQ, tok_body, 0, unroll=False)

        pltpu.sync_copy(rows_v, out_hbm.at[pl.ds(tok, SEQ)])
        return carry

    lax.fori_loop(0, SEQ_PER_W, seq_body, 0, unroll=False)


@jax.jit
def _emb(Xf, table, pe):
    mesh = plsc.VectorSubcoreMesh(core_axis_name="c", subcore_axis_name="s")
    f = functools.partial(
        pl.kernel,
        out_type=jax.ShapeDtypeStruct((TOKENS, EMBED), jnp.float32),
        mesh=mesh,
        scratch_types=[
            pltpu.VMEM((SEQ, EMBED), jnp.float32),
            pltpu.VMEM((SEQ,), jnp.int32),
            pltpu.VMEM((SEQ, EMBED), jnp.float32),
            pltpu.SemaphoreType.DMA,
        ],
    )(_emb_body)
    return f(Xf, table, pe)


def kernel(X, table):
    pe = _pe_slice()
    out = _emb(X.reshape(TOKENS), table, pe)
    return out.reshape(BATCH, SEQ, EMBED)


# speedup vs baseline: 7.2228x; 1.8360x over previous
import functools

import jax
import jax.numpy as jnp
from jax import lax
from jax.experimental import pallas as pl
from jax.experimental.pallas import tpu as pltpu
from jax.experimental.pallas import tpu_sc as plsc

VOCAB = 100000
EMBED = 128
MAX_LEN = 512
BATCH = 1024
SEQ = 200
SCALE = float(EMBED) ** 0.5

NC, NS = 2, 16
NW = NC * NS
TOKENS = BATCH * SEQ
TOK_PER_W = TOKENS // NW
SEQ_PER_W = TOK_PER_W // SEQ
CHUNK = 40
NCHUNK = SEQ // CHUNK
LANES = 16
NBUF = 3


def _pe_slice():
    x = jnp.arange(MAX_LEN, dtype=jnp.float32).reshape((-1, 1)) / jnp.power(
        10000.0, jnp.arange(0, EMBED, 2, dtype=jnp.float32) / EMBED)
    pe = jnp.zeros((MAX_LEN, EMBED), dtype=jnp.float32)
    pe = pe.at[:, 0::2].set(jnp.sin(x))
    pe = pe.at[:, 1::2].set(jnp.sin(x))
    return pe[:SEQ]


def _emb_body(x_hbm, table_hbm, pe_hbm, out_hbm,
              pe_v, idx_v, rows0, rows1, rows2,
              sg0, sg1, sg2, so0, so1, so2):
    wid = lax.axis_index("s") * NC + lax.axis_index("c")
    base = wid * TOK_PER_W
    rows = (rows0, rows1, rows2)
    sg = (sg0, sg1, sg2)
    so = (so0, so1, so2)

    pltpu.sync_copy(pe_hbm, pe_v)
    pltpu.sync_copy(x_hbm.at[pl.ds(base, TOK_PER_W)], idx_v)

    def gather_copies(j, b):
        off = pl.multiple_of(j * SEQ, 8)
        return [pltpu.make_async_copy(
            table_hbm.at[idx_v.at[pl.ds(off + c * CHUNK, CHUNK)]],
            rows[b].at[pl.ds(c * CHUNK, CHUNK)], sg[b])
            for c in range(NCHUNK)]

    def fire(j, b):
        for cp in gather_copies(j, b):
            cp.start()

    def wait_gather(j, b):
        for cp in gather_copies(j, b):
            cp.wait()

    def out_copy(j, b):
        off = pl.multiple_of(base + j * SEQ, 8)
        return pltpu.make_async_copy(rows[b], out_hbm.at[pl.ds(off, SEQ)],
                                     so[b])

    def compute(b):
        r = rows[b]

        def tok_body(t, tc):
            for d in range(EMBED // LANES):
                s = pl.ds(d * LANES, LANES)
                r[t, s] = r[t, s] * SCALE + pe_v[t, s]
            return tc
        lax.fori_loop(0, SEQ, tok_body, 0, unroll=False)

    def step(j, d, first, fire_next=True):
        bf = (d + 1) % NBUF
        if first is None:
            out_copy(j - 2, bf).wait()
        elif not first:
            @pl.when(j >= 2)
            def _():
                out_copy(j - 2, bf).wait()
        if fire_next:
            fire(j + 1, bf)
        wait_gather(j, d)
        compute(d)
        out_copy(j, d).start()

    fire(0, 0)

    def body(k, carry):
        j0 = k * NBUF
        step(j0 + 0, 0, first=False)
        step(j0 + 1, 1, first=False)
        step(j0 + 2, 2, first=None)
        return carry

    lax.fori_loop(0, 10, body, 0, unroll=False)
    step(30, 0, first=None)
    step(31, 1, first=None, fire_next=False)
    out_copy(30, 0).wait()
    out_copy(31, 1).wait()


@jax.jit
def _emb(Xf, table, pe):
    mesh = plsc.VectorSubcoreMesh(core_axis_name="c", subcore_axis_name="s")
    f = functools.partial(
        pl.kernel,
        out_type=jax.ShapeDtypeStruct((TOKENS, EMBED), jnp.float32),
        mesh=mesh,
        scratch_types=[
            pltpu.VMEM((SEQ, EMBED), jnp.float32),
            pltpu.VMEM((TOK_PER_W,), jnp.int32),
            pltpu.VMEM((SEQ, EMBED), jnp.float32),
            pltpu.VMEM((SEQ, EMBED), jnp.float32),
            pltpu.VMEM((SEQ, EMBED), jnp.float32),
            pltpu.SemaphoreType.DMA,
            pltpu.SemaphoreType.DMA,
            pltpu.SemaphoreType.DMA,
            pltpu.SemaphoreType.DMA,
            pltpu.SemaphoreType.DMA,
            pltpu.SemaphoreType.DMA,
        ],
    )(_emb_body)
    return f(Xf, table, pe)


def kernel(X, table):
    pe = _pe_slice()
    out = _emb(X.reshape(TOKENS), table, pe)
    return out.reshape(BATCH, SEQ, EMBED)


# speedup vs baseline: 7.2236x; 1.0001x over previous
import functools

import jax
import jax.numpy as jnp
from jax import lax
from jax.experimental import pallas as pl
from jax.experimental.pallas import tpu as pltpu
from jax.experimental.pallas import tpu_sc as plsc

VOCAB = 100000
EMBED = 128
MAX_LEN = 512
BATCH = 1024
SEQ = 200
SCALE = float(EMBED) ** 0.5

NC, NS = 2, 16
NW = NC * NS
TOKENS = BATCH * SEQ
TOK_PER_W = TOKENS // NW
SEQ_PER_W = TOK_PER_W // SEQ
CHUNK = 40
NCHUNK = SEQ // CHUNK
LANES = 16
NBUF = 3


def _pe_slice():
    x = jnp.arange(MAX_LEN, dtype=jnp.float32).reshape((-1, 1)) / jnp.power(
        10000.0, jnp.arange(0, EMBED, 2, dtype=jnp.float32) / EMBED)
    pe = jnp.zeros((MAX_LEN, EMBED), dtype=jnp.float32)
    pe = pe.at[:, 0::2].set(jnp.sin(x))
    pe = pe.at[:, 1::2].set(jnp.sin(x))
    return pe[:SEQ]


def _emb_body(x_hbm, table_hbm, pe_hbm, out_hbm,
              pe_v, idx_v, rows0, rows1, rows2,
              sg0, sg1, sg2, so0, so1, so2):
    wid = lax.axis_index("s") * NC + lax.axis_index("c")
    base = wid * TOK_PER_W
    rows = (rows0, rows1, rows2)
    sg = (sg0, sg1, sg2)
    so = (so0, so1, so2)

    pltpu.sync_copy(pe_hbm, pe_v)
    pltpu.sync_copy(x_hbm.at[pl.ds(base, TOK_PER_W)], idx_v)

    def gather_copies(j, b):
        off = pl.multiple_of(j * SEQ, 8)
        return [pltpu.make_async_copy(
            table_hbm.at[idx_v.at[pl.ds(off + c * CHUNK, CHUNK)]],
            rows[b].at[pl.ds(c * CHUNK, CHUNK)], sg[b])
            for c in range(NCHUNK)]

    def fire(j, b):
        for cp in gather_copies(j, b):
            cp.start()

    def wait_gather(j, b):
        for cp in gather_copies(j, b):
            cp.wait()

    def out_copy(j, b):
        off = pl.multiple_of(base + j * SEQ, 8)
        return pltpu.make_async_copy(rows[b], out_hbm.at[pl.ds(off, SEQ)],
                                     so[b])

    def compute(b):
        r = rows[b]

        def tok_body(t2, tc):
            for u in range(2):
                t = t2 * 2 + u
                for d in range(EMBED // LANES):
                    s = pl.ds(d * LANES, LANES)
                    r[t, s] = r[t, s] * SCALE + pe_v[t, s]
            return tc
        lax.fori_loop(0, SEQ // 2, tok_body, 0, unroll=False)

    def step(j, d, first, fire_next=True):
        bf = (d + 1) % NBUF
        if first is None:
            out_copy(j - 2, bf).wait()
        elif not first:
            @pl.when(j >= 2)
            def _():
                out_copy(j - 2, bf).wait()
        if fire_next:
            fire(j + 1, bf)
        wait_gather(j, d)
        compute(d)
        out_copy(j, d).start()

    fire(0, 0)

    def body(k, carry):
        j0 = k * NBUF
        step(j0 + 0, 0, first=False)
        step(j0 + 1, 1, first=False)
        step(j0 + 2, 2, first=None)
        return carry

    lax.fori_loop(0, 10, body, 0, unroll=False)
    step(30, 0, first=None)
    step(31, 1, first=None, fire_next=False)
    out_copy(30, 0).wait()
    out_copy(31, 1).wait()


@jax.jit
def _emb(Xf, table, pe):
    mesh = plsc.VectorSubcoreMesh(core_axis_name="c", subcore_axis_name="s")
    f = functools.partial(
        pl.kernel,
        out_type=jax.ShapeDtypeStruct((TOKENS, EMBED), jnp.float32),
        mesh=mesh,
        scratch_types=[
            pltpu.VMEM((SEQ, EMBED), jnp.float32),
            pltpu.VMEM((TOK_PER_W,), jnp.int32),
            pltpu.VMEM((SEQ, EMBED), jnp.float32),
            pltpu.VMEM((SEQ, EMBED), jnp.float32),
            pltpu.VMEM((SEQ, EMBED), jnp.float32),
            pltpu.SemaphoreType.DMA,
            pltpu.SemaphoreType.DMA,
            pltpu.SemaphoreType.DMA,
            pltpu.SemaphoreType.DMA,
            pltpu.SemaphoreType.DMA,
            pltpu.SemaphoreType.DMA,
        ],
    )(_emb_body)
    return f(Xf, table, pe)


def kernel(X, table):
    pe = _pe_slice()
    out = _emb(X.reshape(TOKENS), table, pe)
    return out.reshape(BATCH, SEQ, EMBED)


# speedup vs baseline: 7.2935x; 1.0097x over previous
import functools

import jax
import jax.numpy as jnp
from jax import lax
from jax.experimental import pallas as pl
from jax.experimental.pallas import tpu as pltpu
from jax.experimental.pallas import tpu_sc as plsc

VOCAB = 100000
EMBED = 128
MAX_LEN = 512
BATCH = 1024
SEQ = 200
SCALE = float(EMBED) ** 0.5

NC, NS = 2, 16
NW = NC * NS
TOKENS = BATCH * SEQ
TOK_PER_W = TOKENS // NW
SEQ_PER_W = TOK_PER_W // SEQ
CHUNK = 40
NCHUNK = SEQ // CHUNK
LANES = 16
NBUF = 3


def _pe_slice():
    x = jnp.arange(MAX_LEN, dtype=jnp.float32).reshape((-1, 1)) / jnp.power(
        10000.0, jnp.arange(0, EMBED, 2, dtype=jnp.float32) / EMBED)
    pe = jnp.zeros((MAX_LEN, EMBED), dtype=jnp.float32)
    pe = pe.at[:, 0::2].set(jnp.sin(x))
    pe = pe.at[:, 1::2].set(jnp.sin(x))
    return pe[:SEQ]


def _emb_body(x_hbm, table_hbm, pe_hbm, out_hbm,
              pe_v, idx_v, rows0, rows1, rows2,
              sg0, sg1, sg2, so0, so1, so2):
    wid = lax.axis_index("s") * NC + lax.axis_index("c")
    base = wid * TOK_PER_W
    rows = (rows0, rows1, rows2)
    sg = (sg0, sg1, sg2)
    so = (so0, so1, so2)

    pltpu.sync_copy(x_hbm.at[pl.ds(base, TOK_PER_W)], idx_v)
    pe_cp = pltpu.make_async_copy(pe_hbm, pe_v, so2)
    pe_cp.start()

    def gather_copies(j, b):
        off = pl.multiple_of(j * SEQ, 8)
        return [pltpu.make_async_copy(
            table_hbm.at[idx_v.at[pl.ds(off + c * CHUNK, CHUNK)]],
            rows[b].at[pl.ds(c * CHUNK, CHUNK)], sg[b])
            for c in range(NCHUNK)]

    def fire(j, b):
        for cp in gather_copies(j, b):
            cp.start()

    def wait_gather(j, b):
        for cp in gather_copies(j, b):
            cp.wait()

    def out_copy(j, b):
        off = pl.multiple_of(base + j * SEQ, 8)
        return pltpu.make_async_copy(rows[b], out_hbm.at[pl.ds(off, SEQ)],
                                     so[b])

    def compute(b):
        r = rows[b]

        def tok_body(t2, tc):
            for u in range(2):
                t = t2 * 2 + u
                for d in range(EMBED // LANES):
                    s = pl.ds(d * LANES, LANES)
                    r[t, s] = r[t, s] * SCALE + pe_v[t, s]
            return tc
        lax.fori_loop(0, SEQ // 2, tok_body, 0, unroll=False)

    def step(j, d, first, fire_next=True):
        bf = (d + 1) % NBUF
        if first is None:
            out_copy(j - 2, bf).wait()
        elif not first:
            @pl.when(j >= 2)
            def _():
                out_copy(j - 2, bf).wait()
        if fire_next:
            fire(j + 1, bf)
        wait_gather(j, d)
        compute(d)
        out_copy(j, d).start()

    fire(0, 0)
    pe_cp.wait()

    def body(k, carry):
        j0 = k * NBUF
        step(j0 + 0, 0, first=False)
        step(j0 + 1, 1, first=False)
        step(j0 + 2, 2, first=None)
        return carry

    lax.fori_loop(0, 10, body, 0, unroll=False)
    step(30, 0, first=None)
    step(31, 1, first=None, fire_next=False)
    out_copy(30, 0).wait()
    out_copy(31, 1).wait()


@jax.jit
def _emb(Xf, table, pe):
    mesh = plsc.VectorSubcoreMesh(core_axis_name="c", subcore_axis_name="s")
    f = functools.partial(
        pl.kernel,
        out_type=jax.ShapeDtypeStruct((TOKENS, EMBED), jnp.float32),
        mesh=mesh,
        scratch_types=[
            pltpu.VMEM((SEQ, EMBED), jnp.float32),
            pltpu.VMEM((TOK_PER_W,), jnp.int32),
            pltpu.VMEM((SEQ, EMBED), jnp.float32),
            pltpu.VMEM((SEQ, EMBED), jnp.float32),
            pltpu.VMEM((SEQ, EMBED), jnp.float32),
            pltpu.SemaphoreType.DMA,
            pltpu.SemaphoreType.DMA,
            pltpu.SemaphoreType.DMA,
            pltpu.SemaphoreType.DMA,
            pltpu.SemaphoreType.DMA,
            pltpu.SemaphoreType.DMA,
        ],
    )(_emb_body)
    return f(Xf, table, pe)


def kernel(X, table):
    pe = _pe_slice()
    out = _emb(X.reshape(TOKENS), table, pe)
    return out.reshape(BATCH, SEQ, EMBED)


# speedup vs baseline: 7.3195x; 1.0036x over previous
import functools

import jax
import jax.numpy as jnp
from jax import lax
from jax.experimental import pallas as pl
from jax.experimental.pallas import tpu as pltpu
from jax.experimental.pallas import tpu_sc as plsc

VOCAB = 100000
EMBED = 128
MAX_LEN = 512
BATCH = 1024
SEQ = 200
SCALE = float(EMBED) ** 0.5

NC, NS = 2, 16
NW = NC * NS
TOKENS = BATCH * SEQ
TOK_PER_W = TOKENS // NW
SEQ_PER_W = TOK_PER_W // SEQ
CHUNK = 40
NCHUNK = SEQ // CHUNK
LANES = 16
NBUF = 3


def _pe_slice():
    x = jnp.arange(MAX_LEN, dtype=jnp.float32).reshape((-1, 1)) / jnp.power(
        10000.0, jnp.arange(0, EMBED, 2, dtype=jnp.float32) / EMBED)
    pe = jnp.zeros((MAX_LEN, EMBED), dtype=jnp.float32)
    pe = pe.at[:, 0::2].set(jnp.sin(x))
    pe = pe.at[:, 1::2].set(jnp.sin(x))
    return pe[:SEQ]


def _emb_body(x_hbm, table_hbm, pe_hbm, out_hbm,
              pe_v, idx_v, rows0, rows1, rows2,
              sg0, sg1, sg2, so0, so1, so2):
    wid = lax.axis_index("s") * NC + lax.axis_index("c")
    base = wid * TOK_PER_W
    rows = (rows0, rows1, rows2)
    sg = (sg0, sg1, sg2)
    so = (so0, so1, so2)

    pltpu.sync_copy(x_hbm.at[pl.ds(base, TOK_PER_W)], idx_v)
    pe_cp = pltpu.make_async_copy(pe_hbm, pe_v, so2)
    pe_cp.start()

    def gather_copies(j, b):
        off = pl.multiple_of(j * SEQ, 8)
        return [pltpu.make_async_copy(
            table_hbm.at[idx_v.at[pl.ds(off + c * CHUNK, CHUNK)]],
            rows[b].at[pl.ds(c * CHUNK, CHUNK)], sg[b])
            for c in range(NCHUNK)]

    def fire(j, b):
        for cp in gather_copies(j, b):
            cp.start()

    def wait_gather(j, b):
        for cp in gather_copies(j, b):
            cp.wait()

    def out_copy(j, b):
        off = pl.multiple_of(base + j * SEQ, 8)
        return pltpu.make_async_copy(rows[b], out_hbm.at[pl.ds(off, SEQ)],
                                     so[b])

    def compute(b, lo=0, hi=SEQ):
        r = rows[b]

        def tok_body(t2, tc):
            for u in range(2):
                t = t2 * 2 + u
                for d in range(EMBED // LANES):
                    s = pl.ds(d * LANES, LANES)
                    r[t, s] = r[t, s] * SCALE + pe_v[t, s]
            return tc
        lax.fori_loop(lo // 2, hi // 2, tok_body, 0, unroll=False)

    def step(j, d, first, fire_next=True):
        bf = (d + 1) % NBUF
        if first is None:
            out_copy(j - 2, bf).wait()
        elif not first:
            @pl.when(j >= 2)
            def _():
                out_copy(j - 2, bf).wait()
        if fire_next:
            fire(j + 1, bf)
        wait_gather(j, d)
        compute(d)
        out_copy(j, d).start()

    fire(0, 0)
    pe_cp.wait()

    def body(k, carry):
        j0 = k * NBUF
        step(j0 + 0, 0, first=False)
        step(j0 + 1, 1, first=False)
        step(j0 + 2, 2, first=None)
        return carry

    lax.fori_loop(0, 10, body, 0, unroll=False)
    step(30, 0, first=None)
    out_copy(29, 2).wait()
    wait_gather(31, 1)
    compute(1, 0, 104)
    part_off = pl.multiple_of(base + 31 * SEQ, 8)
    head = pltpu.make_async_copy(
        rows[1].at[pl.ds(0, 104)], out_hbm.at[pl.ds(part_off, 104)], so[1])
    head.start()
    compute(1, 104, SEQ)
    tail = pltpu.make_async_copy(
        rows[1].at[pl.ds(104, 96)], out_hbm.at[pl.ds(part_off + 104, 96)],
        so[1])
    tail.start()
    out_copy(30, 0).wait()
    head.wait()
    tail.wait()


@jax.jit
def _emb(Xf, table, pe):
    mesh = plsc.VectorSubcoreMesh(core_axis_name="c", subcore_axis_name="s")
    f = functools.partial(
        pl.kernel,
        out_type=jax.ShapeDtypeStruct((TOKENS, EMBED), jnp.float32),
        mesh=mesh,
        scratch_types=[
            pltpu.VMEM((SEQ, EMBED), jnp.float32),
            pltpu.VMEM((TOK_PER_W,), jnp.int32),
            pltpu.VMEM((SEQ, EMBED), jnp.float32),
            pltpu.VMEM((SEQ, EMBED), jnp.float32),
            pltpu.VMEM((SEQ, EMBED), jnp.float32),
            pltpu.SemaphoreType.DMA,
            pltpu.SemaphoreType.DMA,
            pltpu.SemaphoreType.DMA,
            pltpu.SemaphoreType.DMA,
            pltpu.SemaphoreType.DMA,
            pltpu.SemaphoreType.DMA,
        ],
    )(_emb_body)
    return f(Xf, table, pe)


def kernel(X, table):
    pe = _pe_slice()
    out = _emb(X.reshape(TOKENS), table, pe)
    return out.reshape(BATCH, SEQ, EMBED)
